# Initial kernel scaffold; baseline (speedup 1.0000x reference)
import jax
import jax.numpy as jnp
from jax import lax
from jax.experimental import pallas as pl
from jax.experimental.pallas import tpu as pltpu
from jax.experimental.pallas import tpu_sc as plsc

_B = 1024
_N = 100000
_E = 128
_H = 4
_K = 6
_HD = _E // _H
_V = 128
_NBLK_PAD = 784
_NV = _NBLK_PAD // 16
_W = 2048
_NT = 49
_RT = 256
_NC = 2
_NS = 16
_RPW = _B // (_NC * _NS)
_NEG = float("-inf")


def _blockmax_body(x_ref, bm_ref):
    j = pl.program_id(1)
    x = x_ref[...]
    col = j * _W + lax.broadcasted_iota(jnp.int32, (_RT, _W), 1)
    x = jnp.where(col < _N, x, _NEG)
    bm_ref[...] = jnp.max(x.reshape(_RT, _W // _V, _V), axis=2)


def _blockmax(logits):
    return pl.pallas_call(
        _blockmax_body,
        grid=(_B // _RT, _NT),
        in_specs=[pl.BlockSpec((_RT, _W), lambda i, j: (i, j))],
        out_specs=pl.BlockSpec((_RT, _W // _V), lambda i, j: (i, j)),
        out_shape=jax.ShapeDtypeStruct((_B, _NBLK_PAD), jnp.float32),
        compiler_params=pltpu.CompilerParams(
            dimension_semantics=("parallel", "arbitrary")),
    )(logits)


def _sc_body(bm_hbm, logits_hbm, bank_hbm, sel_hbm,
             bm_v, cand_v, gidx_v, idx_v, rows_v, sem_a, sem_b):
    wid = lax.axis_index("s") * _NC + lax.axis_index("c")
    lane = lax.iota(jnp.int32, 16)
    neg = jnp.full((16,), _NEG, jnp.float32)

    def row_body(i, carry):
        r = wid * _RPW + i
        pltpu.sync_copy(bm_hbm.at[r], bm_v)

        starts = []
        for _ in range(_K):
            def smax(iv, acc):
                return jnp.maximum(acc, bm_v[pl.ds(iv * 16, 16)])
            acc = lax.fori_loop(0, _NV, smax, neg)
            m = jnp.max(acc)

            def sidx(iv, best):
                v = bm_v[pl.ds(iv * 16, 16)]
                c = jnp.where(v == m, iv * 16 + lane, jnp.int32(2 * _NBLK_PAD))
                return jnp.minimum(best, jnp.min(c))
            blk = lax.fori_loop(0, _NV, sidx, jnp.int32(2 * _NBLK_PAD))
            plsc.store_scatter(bm_v, [jnp.broadcast_to(blk, (16,))], neg,
                               mask=lane == 0)
            starts.append(jnp.minimum(blk * _V, _N - _V))

        cps = [pltpu.async_copy(logits_hbm.at[r, pl.ds(starts[kk], _V)],
                                cand_v.at[pl.ds(kk * _V, _V)], sem_a)
               for kk in range(_K)]
        for cp in cps:
            cp.wait()
        for kk in range(_K):
            for iv in range(_V // 16):
                gidx_v[pl.ds(kk * _V + iv * 16, 16)] = \
                    starts[kk] + (iv * 16 + lane)

        nvv = _K * _V // 16
        chosen = []
        for _ in range(_K):
            prevs = list(chosen)

            def dead_mask(gi):
                d = gi == prevs[0]
                for p in prevs[1:]:
                    d = d | (gi == p)
                return d

            def smax2(iv, acc):
                v = cand_v[pl.ds(iv * 16, 16)]
                if prevs:
                    v = jnp.where(dead_mask(gidx_v[pl.ds(iv * 16, 16)]),
                                  _NEG, v)
                return jnp.maximum(acc, v)
            acc = lax.fori_loop(0, nvv, smax2, neg)
            m = jnp.max(acc)

            def sidx2(iv, best):
                v = cand_v[pl.ds(iv * 16, 16)]
                gi = gidx_v[pl.ds(iv * 16, 16)]
                ok = v == m
                if prevs:
                    ok = ok & (~dead_mask(gi))
                c = jnp.where(ok, gi, jnp.int32(2 * _N))
                return jnp.minimum(best, jnp.min(c))
            chosen.append(lax.fori_loop(0, nvv, sidx2, jnp.int32(2 * _N)))

        gvec = jnp.zeros((16,), jnp.int32)
        for kk in range(_K):
            gvec = jnp.where(lane == kk, chosen[kk], gvec)
        idx_v[...] = gvec
        pltpu.async_copy(bank_hbm.at[idx_v], rows_v, sem_b).wait()
        pltpu.sync_copy(rows_v.at[pl.ds(0, _K)], sel_hbm.at[r])
        return carry

    lax.fori_loop(0, _RPW, row_body, jnp.int32(0))


def _sc_topk_gather(bm, logits, bank):
    mesh = plsc.VectorSubcoreMesh(core_axis_name="c", subcore_axis_name="s",
                                  num_cores=_NC, num_subcores=_NS)
    f = pl.kernel(
        _sc_body,
        out_type=jax.ShapeDtypeStruct((_B, _K, _E), jnp.float32),
        mesh=mesh,
        scratch_types=[
            pltpu.VMEM((_NBLK_PAD,), jnp.float32),
            pltpu.VMEM((_K * _V,), jnp.float32),
            pltpu.VMEM((_K * _V,), jnp.int32),
            pltpu.VMEM((16,), jnp.int32),
            pltpu.VMEM((16, _E), jnp.float32),
            pltpu.SemaphoreType.DMA,
            pltpu.SemaphoreType.DMA,
        ],
    )
    return f(bm, logits, bank)


def _attn_body(q_ref, sel_ref, wq_ref, bq_ref, wp_ref, bp_ref,
               inw_ref, inb_ref, outw_ref, outb_ref, ctx_ref, aw_ref):
    f32 = jnp.float32

    def dot_t(a, b):
        return lax.dot_general(a, b, (((1,), (1,)), ((), ())),
                               preferred_element_type=f32)

    q = q_ref[...]
    aq = dot_t(q, wq_ref[...]) + bq_ref[...]
    qp = dot_t(aq, inw_ref[0:_E, :]) + inb_ref[0:1, :]

    d_i = lax.broadcasted_iota(jnp.int32, (_E, _H), 0)
    h_i = lax.broadcasted_iota(jnp.int32, (_E, _H), 1)
    sel_m = (d_i // _HD == h_i).astype(f32)
    scale = _HD ** -0.5

    ts, vs = [], []
    for j in range(_K):
        kv = dot_t(sel_ref[:, j, :], wp_ref[...]) + bp_ref[...]
        kp = dot_t(kv, inw_ref[_E:2 * _E, :]) + inb_ref[1:2, :]
        vp = dot_t(kv, inw_ref[2 * _E:3 * _E, :]) + inb_ref[2:3, :]
        t = lax.dot_general(qp * kp, sel_m, (((1,), (0,)), ((), ())),
                            preferred_element_type=f32) * scale
        ts.append(t)
        vs.append(vp)

    m = ts[0]
    for t in ts[1:]:
        m = jnp.maximum(m, t)
    es = [jnp.exp(t - m) for t in ts]
    z = es[0]
    for e in es[1:]:
        z = z + e
    ws = [e / z for e in es]

    aw = jnp.concatenate(
        [jnp.sum(w, axis=1, keepdims=True) for w in ws], axis=1) * (1.0 / _H)

    ctx = jnp.zeros_like(qp)
    for j in range(_K):
        wexp = dot_t(ws[j], sel_m)
        ctx = ctx + wexp * vs[j]
    ctx_ref[...] = dot_t(ctx, outw_ref[...]) + outb_ref[...]
    aw_ref[...] = aw


def _attn(query, sel, W_q, b_q, W_p, b_p, inw, inb, outw, outb):
    def full(shape):
        return pl.BlockSpec(shape, lambda i: tuple(0 for _ in shape))
    return pl.pallas_call(
        _attn_body,
        grid=(_B // _RT,),
        in_specs=[
            pl.BlockSpec((_RT, _E), lambda i: (i, 0)),
            pl.BlockSpec((_RT, _K, _E), lambda i: (i, 0, 0)),
            full((_E, _E)), full((1, _E)),
            full((_E, _E)), full((1, _E)),
            full((3 * _E, _E)), full((3, _E)),
            full((_E, _E)), full((1, _E)),
        ],
        out_specs=[
            pl.BlockSpec((_RT, _E), lambda i: (i, 0)),
            pl.BlockSpec((_RT, _K), lambda i: (i, 0)),
        ],
        out_shape=[
            jax.ShapeDtypeStruct((_B, _E), jnp.float32),
            jax.ShapeDtypeStruct((_B, _K), jnp.float32),
        ],
    )(query, sel, W_q, b_q.reshape(1, _E), W_p, b_p.reshape(1, _E),
      inw, inb.reshape(3, _E), outw, outb.reshape(1, _E))


def kernel(query, prototype_bank, prototype_logits, W_q_proj, b_q_proj,
           W_p_proj, b_p_proj, in_proj_w, in_proj_b, out_proj_w, out_proj_b):
    bm = _blockmax(prototype_logits)
    sel = _sc_topk_gather(bm, prototype_logits, prototype_bank)
    return _attn(query, sel, W_q_proj, b_q_proj, W_p_proj, b_p_proj,
                 in_proj_w, in_proj_b, out_proj_w, out_proj_b)


# speedup vs baseline: 1.5680x; 1.5680x over previous
import jax
import jax.numpy as jnp
from jax import lax
from jax.experimental import pallas as pl
from jax.experimental.pallas import tpu as pltpu
from jax.experimental.pallas import tpu_sc as plsc

_B = 1024
_N = 100000
_E = 128
_H = 4
_K = 6
_HD = _E // _H
_V = 128
_NBLK_PAD = 896
_NV = _NBLK_PAD // 16
_W = 16384
_NT = 7
_RTA = 64
_RT = 256
_TAIL = 781
_TS = _TAIL * _V - 6 * _W
_NC = 2
_NS = 16
_RPW = _B // (_NC * _NS)
_NEG = float("-inf")


def _blockmax_body(x_ref, bm_ref, aux_ref):
    j = pl.program_id(1)
    x = x_ref[...]
    col = j * _W + lax.broadcasted_iota(jnp.int32, (_RTA, _W), 1)
    x = jnp.where(col < _N, x, _NEG)
    bm_ref[...] = jnp.max(x.reshape(_RTA, _W // _V, _V), axis=2)

    @pl.when(j == _NT - 1)
    def _():
        aux_ref[...] = x[:, _TS:_TS + _V]


def _blockmax(logits):
    return pl.pallas_call(
        _blockmax_body,
        grid=(_B // _RTA, _NT),
        in_specs=[pl.BlockSpec((_RTA, _W), lambda i, j: (i, j))],
        out_specs=[
            pl.BlockSpec((_RTA, _W // _V), lambda i, j: (i, j)),
            pl.BlockSpec((_RTA, _V), lambda i, j: (i, 0)),
        ],
        out_shape=[
            jax.ShapeDtypeStruct((_B, _NBLK_PAD), jnp.float32),
            jax.ShapeDtypeStruct((_B, _V), jnp.float32),
        ],
        compiler_params=pltpu.CompilerParams(
            dimension_semantics=("parallel", "arbitrary")),
    )(logits)


def _sc_body(bm_hbm, logits_hbm, aux_hbm, bank_hbm, sel_hbm,
             bm_s, cand_s, gidx_v, idx_v, rows_v, sem_a, sem_b):
    wid = lax.axis_index("s") * _NC + lax.axis_index("c")
    lane = lax.iota(jnp.int32, 16)
    neg = jnp.full((16,), _NEG, jnp.float32)
    big = jnp.full((16,), 2 * _N, jnp.int32)
    nvv = _K * _V // 16

    def _htree(vec, op):
        xs = [vec[l] for l in range(16)]
        while len(xs) > 1:
            nxt = [op(xs[i], xs[i + 1]) for i in range(0, len(xs) - 1, 2)]
            if len(xs) % 2:
                nxt.append(xs[-1])
            xs = nxt
        return xs[0]

    def hmax_f(vec):
        return _htree(vec, jnp.maximum)

    def hmin_i(vec):
        return _htree(vec, jnp.minimum)

    def slab_body(sb, carry0):
        rbase = pl.multiple_of(wid * _RPW + sb * 8, 8)
        pltpu.sync_copy(bm_hbm.at[pl.ds(rbase, 8)], bm_s)

        def row_body(rm, carry):
            starts = []
            blks = []
            for kk in range(_K):
                prevb = list(blks)

                def smax(iv, acc):
                    v = bm_s[rm, pl.ds(iv * 16, 16)]
                    ids = iv * 16 + lane
                    for p in prevb:
                        v = jnp.where(ids == p, _NEG, v)
                    return jnp.maximum(acc, v)
                acc = lax.fori_loop(0, _NV, smax, neg)
                m = hmax_f(acc)

                def sidx(iv, best):
                    v = bm_s[rm, pl.ds(iv * 16, 16)]
                    ids = iv * 16 + lane
                    c = jnp.where(v == m, ids, big)
                    for p in prevb:
                        c = jnp.where(ids == p, big, c)
                    return jnp.minimum(best, c)
                bestv = lax.fori_loop(0, _NV, sidx, big)
                blk = hmin_i(bestv)
                blks.append(blk)
                starts.append(pl.multiple_of(blk * _V, _V))

                @pl.when(blk < _TAIL)
                def _():
                    pltpu.async_copy(
                        logits_hbm.at[pl.ds(rbase, 8), pl.ds(starts[kk], _V)],
                        cand_s.at[pl.ds(0, 8), pl.ds(kk * _V, _V)], sem_a)

                @pl.when(blk >= _TAIL)
                def _():
                    pltpu.async_copy(
                        aux_hbm.at[pl.ds(rbase, 8)],
                        cand_s.at[pl.ds(0, 8), pl.ds(kk * _V, _V)], sem_a)

            for kk in range(_K):
                pltpu.make_async_copy(
                    logits_hbm.at[pl.ds(0, 8), pl.ds(0, _V)],
                    cand_s.at[pl.ds(0, 8), pl.ds(kk * _V, _V)], sem_a).wait()

            for kk in range(_K):
                for iv in range(_V // 16):
                    gidx_v[pl.ds(kk * _V + iv * 16, 16)] = \
                        starts[kk] + (iv * 16 + lane)

            chosen = []
            for _ in range(_K):
                prevs = list(chosen)

                def smax2(iv, acc):
                    v = cand_s[rm, pl.ds(iv * 16, 16)]
                    if prevs:
                        gi = gidx_v[pl.ds(iv * 16, 16)]
                        for p in prevs:
                            v = jnp.where(gi == p, _NEG, v)
                    return jnp.maximum(acc, v)
                acc = lax.fori_loop(0, nvv, smax2, neg)
                m = hmax_f(acc)

                def sidx2(iv, best):
                    v = cand_s[rm, pl.ds(iv * 16, 16)]
                    gi = gidx_v[pl.ds(iv * 16, 16)]
                    c = jnp.where(v == m, gi, big)
                    for p in prevs:
                        c = jnp.where(gi == p, big, c)
                    return jnp.minimum(best, c)
                bestv2 = lax.fori_loop(0, nvv, sidx2, big)
                chosen.append(hmin_i(bestv2))

            gvec = jnp.zeros((16,), jnp.int32)
            for kk in range(_K):
                gvec = jnp.where(lane == kk, chosen[kk], gvec)
            idx_v[...] = gvec
            pltpu.async_copy(bank_hbm.at[idx_v], rows_v, sem_b).wait()
            pltpu.sync_copy(rows_v.at[pl.ds(0, 8)], sel_hbm.at[rbase + rm])
            return carry

        lax.fori_loop(0, 8, row_body, jnp.int32(0))
        return carry0

    lax.fori_loop(0, _RPW // 8, slab_body, jnp.int32(0))


def _sc_topk_gather(bm, aux, logits, bank):
    mesh = plsc.VectorSubcoreMesh(core_axis_name="c", subcore_axis_name="s",
                                  num_cores=_NC, num_subcores=_NS)
    f = pl.kernel(
        _sc_body,
        out_type=jax.ShapeDtypeStruct((_B, 8, _E), jnp.float32),
        mesh=mesh,
        scratch_types=[
            pltpu.VMEM((8, _NBLK_PAD), jnp.float32),
            pltpu.VMEM((8, _K * _V), jnp.float32),
            pltpu.VMEM((_K * _V,), jnp.int32),
            pltpu.VMEM((16,), jnp.int32),
            pltpu.VMEM((16, _E), jnp.float32),
            pltpu.SemaphoreType.DMA,
            pltpu.SemaphoreType.DMA,
        ],
    )
    return f(bm, logits, aux, bank)


def _attn_body(q_ref, sel_ref, wq_ref, bq_ref, wp_ref, bp_ref,
               inw_ref, inb_ref, outw_ref, outb_ref, ctx_ref, aw_ref):
    f32 = jnp.float32

    def dot_t(a, b):
        return lax.dot_general(a, b, (((1,), (1,)), ((), ())),
                               preferred_element_type=f32)

    q = q_ref[...]
    aq = dot_t(q, wq_ref[...]) + bq_ref[...]
    qp = dot_t(aq, inw_ref[0:_E, :]) + inb_ref[0:1, :]

    d_i = lax.broadcasted_iota(jnp.int32, (_E, _H), 0)
    h_i = lax.broadcasted_iota(jnp.int32, (_E, _H), 1)
    sel_m = (d_i // _HD == h_i).astype(f32)
    scale = _HD ** -0.5

    ts, vs = [], []
    for j in range(_K):
        kv = dot_t(sel_ref[:, j, :], wp_ref[...]) + bp_ref[...]
        kp = dot_t(kv, inw_ref[_E:2 * _E, :]) + inb_ref[1:2, :]
        vp = dot_t(kv, inw_ref[2 * _E:3 * _E, :]) + inb_ref[2:3, :]
        t = lax.dot_general(qp * kp, sel_m, (((1,), (0,)), ((), ())),
                            preferred_element_type=f32) * scale
        ts.append(t)
        vs.append(vp)

    m = ts[0]
    for t in ts[1:]:
        m = jnp.maximum(m, t)
    es = [jnp.exp(t - m) for t in ts]
    z = es[0]
    for e in es[1:]:
        z = z + e
    ws = [e / z for e in es]

    aw = jnp.concatenate(
        [jnp.sum(w, axis=1, keepdims=True) for w in ws], axis=1) * (1.0 / _H)

    ctx = jnp.zeros_like(qp)
    for j in range(_K):
        wexp = dot_t(ws[j], sel_m)
        ctx = ctx + wexp * vs[j]
    ctx_ref[...] = dot_t(ctx, outw_ref[...]) + outb_ref[...]
    aw_ref[...] = aw


def _attn(query, sel, W_q, b_q, W_p, b_p, inw, inb, outw, outb):
    def full(shape):
        return pl.BlockSpec(shape, lambda i: tuple(0 for _ in shape))
    return pl.pallas_call(
        _attn_body,
        grid=(_B // _RT,),
        in_specs=[
            pl.BlockSpec((_RT, _E), lambda i: (i, 0)),
            pl.BlockSpec((_RT, 8, _E), lambda i: (i, 0, 0)),
            full((_E, _E)), full((1, _E)),
            full((_E, _E)), full((1, _E)),
            full((3 * _E, _E)), full((3, _E)),
            full((_E, _E)), full((1, _E)),
        ],
        out_specs=[
            pl.BlockSpec((_RT, _E), lambda i: (i, 0)),
            pl.BlockSpec((_RT, _K), lambda i: (i, 0)),
        ],
        out_shape=[
            jax.ShapeDtypeStruct((_B, _E), jnp.float32),
            jax.ShapeDtypeStruct((_B, _K), jnp.float32),
        ],
    )(query, sel, W_q, b_q.reshape(1, _E), W_p, b_p.reshape(1, _E),
      inw, inb.reshape(3, _E), outw, outb.reshape(1, _E))


def kernel(query, prototype_bank, prototype_logits, W_q_proj, b_q_proj,
           W_p_proj, b_p_proj, in_proj_w, in_proj_b, out_proj_w, out_proj_b):
    bm, aux = _blockmax(prototype_logits)
    sel = _sc_topk_gather(bm, aux, prototype_logits, prototype_bank)
    return _attn(query, sel, W_q_proj, b_q_proj, W_p_proj, b_p_proj,
                 in_proj_w, in_proj_b, out_proj_w, out_proj_b)
